# baseline (speedup 1.0000x reference)
import functools

import jax
import jax.numpy as jnp
from jax import lax
from jax.experimental import pallas as pl
from jax.experimental.pallas import tpu as pltpu
from jax.experimental.pallas import tpu_sc as plsc

H = 128
NRBF = 32
NNODES = 10000
NEDGES = 160000
CUT_UPPER = 5.0

EB = 256
NB = 256
NBLK = (NNODES + NB - 1) // NB
NPAD = NBLK * NB
CT = 672
S = CT * EB
FEATW = 128

SC_NW = 32
SC_K = 128
EPAD = 163840
SC_ITERS = EPAD // (SC_NW * SC_K)


def _node_kernel(zf_ref, emb_ref, w1t_ref, w2t_ref, zw1_ref, zw2_ref):
    zrow = zf_ref[0]
    blk = zrow.shape[-1]
    q = lax.broadcasted_iota(jnp.int32, (H, blk), 0).astype(jnp.float32)
    oht = (q == zrow).astype(jnp.float32)
    z_emb = lax.dot_general(oht, emb_ref[...],
                            (((0,), (0,)), ((), ())),
                            preferred_element_type=jnp.float32)
    zw1_ref[...] = jnp.dot(z_emb, w1t_ref[...], preferred_element_type=jnp.float32)
    zw2_ref[...] = jnp.dot(z_emb, w2t_ref[...], preferred_element_type=jnp.float32)


def _node_precompute(z_f, emb_w, w1t, w2t):
    blk = 512
    nblk = NPAD // blk
    zf3 = z_f.reshape(nblk, 1, blk)
    return pl.pallas_call(
        _node_kernel,
        grid=(nblk,),
        in_specs=[
            pl.BlockSpec((1, 1, blk), lambda i: (i, 0, 0)),
            pl.BlockSpec((H, H), lambda i: (0, 0)),
            pl.BlockSpec((H, H), lambda i: (0, 0)),
            pl.BlockSpec((H, H), lambda i: (0, 0)),
        ],
        out_specs=[
            pl.BlockSpec((blk, H), lambda i: (i, 0)),
            pl.BlockSpec((blk, H), lambda i: (i, 0)),
        ],
        out_shape=[
            jax.ShapeDtypeStruct((NPAD, H), jnp.float32),
            jax.ShapeDtypeStruct((NPAD, H), jnp.float32),
        ],
    )(zf3, emb_w, w1t, w2t)


def _sc_scatter_body(feat_hbm, slotm_hbm, dstm_hbm, srcm_hbm, zw1_hbm, zw2_hbm,
                     feats_out, zij_out,
                     slot_v, dst_v, src_v, fb0, fb1, z1b0, z1b1, z2b0, z2b1,
                     gsem0, gsem1, ssem0, ssem1):
    nc = 2
    wid = lax.axis_index("s") * nc + lax.axis_index("c")
    rowbase = wid * SC_ITERS
    base = wid * (SC_ITERS * SC_K)
    pltpu.sync_copy(slotm_hbm.at[pl.ds(rowbase, SC_ITERS)], slot_v)
    pltpu.sync_copy(dstm_hbm.at[pl.ds(rowbase, SC_ITERS)], dst_v)
    pltpu.sync_copy(srcm_hbm.at[pl.ds(rowbase, SC_ITERS)], src_v)

    fb = (fb0, fb1)
    z1 = (z1b0, z1b1)
    z2 = (z2b0, z2b1)
    gs = (gsem0, gsem1)
    ss = (ssem0, ssem1)

    def fire_loads(j, b):
        off = base + j * SC_K
        pltpu.async_copy(feat_hbm.at[pl.ds(off, SC_K)], fb[b], gs[b])
        pltpu.async_copy(zw1_hbm.at[dst_v.at[j]], z1[b], gs[b])
        pltpu.async_copy(zw2_hbm.at[src_v.at[j]], z2[b], gs[b])

    def drain_loads(b):
        pltpu.make_async_copy(feat_hbm.at[pl.ds(0, SC_K)], fb[b], gs[b]).wait()
        pltpu.make_async_copy(zw1_hbm.at[pl.ds(0, SC_K)], z1[b], gs[b]).wait()
        pltpu.make_async_copy(zw2_hbm.at[pl.ds(0, SC_K)], z2[b], gs[b]).wait()

    def drain_stores(b):
        pltpu.make_async_copy(feat_hbm.at[pl.ds(0, SC_K)], fb[b], ss[b]).wait()
        pltpu.make_async_copy(zw1_hbm.at[pl.ds(0, SC_K)], z1[b], ss[b]).wait()

    def do_store(j, b):
        z1b = z1[b]
        z2b = z2[b]

        def row(r, carry):
            for c in range(H // 16):
                sl = pl.ds(c * 16, 16)
                z1b[r, sl] = z1b[r, sl] + z2b[r, sl]
            return carry

        lax.fori_loop(0, SC_K, row, 0, unroll=False)
        pltpu.async_copy(fb[b], feats_out.at[slot_v.at[j]], ss[b])
        pltpu.async_copy(z1b, zij_out.at[slot_v.at[j]], ss[b])

    def body(g, carry):
        for b in (0, 1):
            @pl.when(g > 0)
            def _():
                drain_stores(b)
            fire_loads(2 * g + b, b)
        for b in (0, 1):
            drain_loads(b)
            do_store(2 * g + b, b)
        return carry

    lax.fori_loop(0, SC_ITERS // 2, body, 0, unroll=False)
    drain_stores(0)
    drain_stores(1)


def _sc_scatter(feat, slotm, dstm, srcm, zw1, zw2):
    mesh = plsc.VectorSubcoreMesh(core_axis_name="c", subcore_axis_name="s")
    fn = functools.partial(
        pl.kernel, mesh=mesh,
        out_type=[
            jax.ShapeDtypeStruct((S, FEATW), jnp.float32),
            jax.ShapeDtypeStruct((S, H), jnp.float32),
        ],
        scratch_types=[
            pltpu.VMEM((SC_ITERS, SC_K), jnp.int32),
            pltpu.VMEM((SC_ITERS, SC_K), jnp.int32),
            pltpu.VMEM((SC_ITERS, SC_K), jnp.int32),
            pltpu.VMEM((SC_K, FEATW), jnp.float32),
            pltpu.VMEM((SC_K, FEATW), jnp.float32),
            pltpu.VMEM((SC_K, H), jnp.float32),
            pltpu.VMEM((SC_K, H), jnp.float32),
            pltpu.VMEM((SC_K, H), jnp.float32),
            pltpu.VMEM((SC_K, H), jnp.float32),
            pltpu.SemaphoreType.DMA,
            pltpu.SemaphoreType.DMA,
            pltpu.SemaphoreType.DMA,
            pltpu.SemaphoreType.DMA,
        ],
    )(_sc_scatter_body)
    return fn(feat, slotm, dstm, srcm, zw1, zw2)


def _edge_kernel(nb_map_ref, first_ref, vcnt_ref, feat_ref, zij_ref,
                 d1t_ref, d2t_ref, d3t_ref, db_ref, eb2b_ref, t_ref):
    c = pl.program_id(0)
    feat = feat_ref[0]
    attr = feat[:, :NRBF]
    vx = feat[:, NRBF:NRBF + 1]
    vy = feat[:, NRBF + 1:NRBF + 2]
    vz = feat[:, NRBF + 2:NRBF + 3]
    w = feat[:, NRBF + 3:NRBF + 4]
    dl = feat[:, NRBF + 4:NRBF + 5]

    zij = zij_ref[0] + eb2b_ref[0]
    cut = 0.5 * (jnp.cos(w * (jnp.pi / CUT_UPPER)) + 1.0)
    cut = cut * (w < CUT_UPPER).astype(jnp.float32)
    cz = cut * zij

    d1 = jnp.dot(attr, d1t_ref[...], preferred_element_type=jnp.float32) + db_ref[0, 0:1]
    d2 = jnp.dot(attr, d2t_ref[...], preferred_element_type=jnp.float32) + db_ref[1, 0:1]
    d3 = jnp.dot(attr, d3t_ref[...], preferred_element_type=jnp.float32) + db_ref[2, 0:1]
    ci = d1 * cz
    ca = d2 * cz
    cs = d3 * cz

    tr3 = (vx * vx + vy * vy + vz * vz) * (1.0 / 3.0)
    payload = jnp.concatenate([
        ci,
        ca * vx, ca * vy, ca * vz,
        cs * (vx * vx - tr3), cs * (vx * vy), cs * (vx * vz),
        cs * (vy * vy - tr3), cs * (vy * vz),
    ], axis=1)

    ridx = lax.broadcasted_iota(jnp.int32, (EB, 1), 0)
    valid = ridx < vcnt_ref[c]
    payload = jnp.where(valid, payload, 0.0)

    cols = lax.broadcasted_iota(jnp.int32, (EB, NB), 1).astype(jnp.float32)
    onehot = (cols == dl).astype(jnp.float32)
    contrib = lax.dot_general(onehot, payload,
                              (((0,), (0,)), ((), ())),
                              preferred_element_type=jnp.float32)

    @pl.when(first_ref[c] == 1)
    def _():
        t_ref[...] = jnp.zeros_like(t_ref)

    t_ref[...] += contrib


def _edge_accumulate(feat_s, zij_s, chunk_nb, chunk_first, vcnt,
                     d1t, d2t, d3t, dbias, eb2b):
    grid_spec = pltpu.PrefetchScalarGridSpec(
        num_scalar_prefetch=3,
        grid=(CT,),
        in_specs=[
            pl.BlockSpec((1, EB, FEATW), lambda c, nbm, fst, vc: (c, 0, 0)),
            pl.BlockSpec((1, EB, H), lambda c, nbm, fst, vc: (c, 0, 0)),
            pl.BlockSpec((NRBF, H), lambda c, nbm, fst, vc: (0, 0)),
            pl.BlockSpec((NRBF, H), lambda c, nbm, fst, vc: (0, 0)),
            pl.BlockSpec((NRBF, H), lambda c, nbm, fst, vc: (0, 0)),
            pl.BlockSpec((3, H), lambda c, nbm, fst, vc: (0, 0)),
            pl.BlockSpec((1, H), lambda c, nbm, fst, vc: (0, 0)),
        ],
        out_specs=pl.BlockSpec((NB, 9 * H), lambda c, nbm, fst, vc: (nbm[c], 0)),
    )
    return pl.pallas_call(
        _edge_kernel,
        grid_spec=grid_spec,
        out_shape=jax.ShapeDtypeStruct((NPAD, 9 * H), jnp.float32),
    )(chunk_nb, chunk_first, vcnt,
      feat_s.reshape(CT, EB, FEATW), zij_s.reshape(CT, EB, H),
      d1t, d2t, d3t, dbias, eb2b)


def _silu(x):
    return x / (1.0 + jnp.exp(-x))


def _final_kernel(t_ref, lng_ref, lnb_ref, ls0t_ref, ls0b_ref, ls1t_ref,
                  ls1b_ref, lt0t_ref, lt1t_ref, lt2t_ref, out_ref):
    t = t_ref[...]
    t0 = t[:, 0:H]
    a1 = t[:, H:2 * H]
    a2 = t[:, 2 * H:3 * H]
    a3 = t[:, 3 * H:4 * H]
    s1 = t[:, 4 * H:5 * H]
    s2 = t[:, 5 * H:6 * H]
    s3 = t[:, 6 * H:7 * H]
    s4 = t[:, 7 * H:8 * H]
    s5 = t[:, 8 * H:9 * H]

    nrm = (3.0 * t0 * t0
           + 2.0 * (a1 * a1 + a2 * a2 + a3 * a3)
           + s1 * s1 + s4 * s4 + (s1 + s4) * (s1 + s4)
           + 2.0 * (s2 * s2 + s3 * s3 + s5 * s5))

    mu = jnp.mean(nrm, axis=1, keepdims=True)
    dn = nrm - mu
    var = jnp.mean(dn * dn, axis=1, keepdims=True)
    nh = dn * lax.rsqrt(var + 1e-5) * lng_ref[0] + lnb_ref[0]

    h1 = _silu(jnp.dot(nh, ls0t_ref[...], preferred_element_type=jnp.float32)
               + ls0b_ref[0])
    h2 = _silu(jnp.dot(h1, ls1t_ref[...], preferred_element_type=jnp.float32)
               + ls1b_ref[0])
    f0 = h2[:, 0:H]
    f1 = h2[:, H:2 * H]
    f2 = h2[:, 2 * H:3 * H]

    u0 = jnp.dot(t0, lt0t_ref[...], preferred_element_type=jnp.float32) * f0
    ua1 = jnp.dot(a1, lt1t_ref[...], preferred_element_type=jnp.float32) * f1
    ua2 = jnp.dot(a2, lt1t_ref[...], preferred_element_type=jnp.float32) * f1
    ua3 = jnp.dot(a3, lt1t_ref[...], preferred_element_type=jnp.float32) * f1
    us1 = jnp.dot(s1, lt2t_ref[...], preferred_element_type=jnp.float32) * f2
    us2 = jnp.dot(s2, lt2t_ref[...], preferred_element_type=jnp.float32) * f2
    us3 = jnp.dot(s3, lt2t_ref[...], preferred_element_type=jnp.float32) * f2
    us4 = jnp.dot(s4, lt2t_ref[...], preferred_element_type=jnp.float32) * f2
    us5 = jnp.dot(s5, lt2t_ref[...], preferred_element_type=jnp.float32) * f2

    out_ref[0] = u0 + us1
    out_ref[1] = -ua3 + us2
    out_ref[2] = ua2 + us3
    out_ref[3] = ua3 + us2
    out_ref[4] = u0 + us4
    out_ref[5] = -ua1 + us5
    out_ref[6] = -ua2 + us3
    out_ref[7] = ua1 + us5
    out_ref[8] = u0 - us1 - us4


def _final_stage(t_acc, ln_g, ln_b, ls0t, ls0b, ls1t, ls1b, lt0t, lt1t, lt2t):
    blk = 512
    nblk = NPAD // blk
    return pl.pallas_call(
        _final_kernel,
        grid=(nblk,),
        in_specs=[
            pl.BlockSpec((blk, 9 * H), lambda i: (i, 0)),
            pl.BlockSpec((1, H), lambda i: (0, 0)),
            pl.BlockSpec((1, H), lambda i: (0, 0)),
            pl.BlockSpec((H, 2 * H), lambda i: (0, 0)),
            pl.BlockSpec((1, 2 * H), lambda i: (0, 0)),
            pl.BlockSpec((2 * H, 3 * H), lambda i: (0, 0)),
            pl.BlockSpec((1, 3 * H), lambda i: (0, 0)),
            pl.BlockSpec((H, H), lambda i: (0, 0)),
            pl.BlockSpec((H, H), lambda i: (0, 0)),
            pl.BlockSpec((H, H), lambda i: (0, 0)),
        ],
        out_specs=pl.BlockSpec((9, blk, H), lambda i: (0, i, 0)),
        out_shape=jax.ShapeDtypeStruct((9, NPAD, H), jnp.float32),
    )(t_acc, ln_g, ln_b, ls0t, ls0b, ls1t, ls1b, lt0t, lt1t, lt2t)


def kernel(z, edge_index, edge_weight, edge_vec_norm, edge_attr, emb_w,
           emb2_w, emb2_b, dp1_w, dp1_b, dp2_w, dp2_b, dp3_w, dp3_b,
           lt0_w, lt1_w, lt2_w, ls0_w, ls0_b, ls1_w, ls1_b, ln_g, ln_b):
    f32 = jnp.float32
    i32 = jnp.int32
    dst = edge_index[0].astype(i32)
    src = edge_index[1].astype(i32)

    bucket = dst // NB
    oh = (bucket[:, None] == jnp.arange(NBLK, dtype=i32)[None, :])
    csum = jnp.cumsum(oh.astype(i32), axis=0)
    rank = jnp.sum(jnp.where(oh, csum, 0), axis=1) - 1
    cnt = csum[-1]
    chunks = jnp.maximum((cnt + EB - 1) // EB, 1)
    chunk_start = jnp.concatenate([jnp.zeros((1,), i32),
                                   jnp.cumsum(chunks).astype(i32)])
    cidx = jnp.arange(CT, dtype=i32)
    chunk_nb = jnp.clip(
        jnp.searchsorted(chunk_start, cidx, side="right").astype(i32) - 1,
        0, NBLK - 1)
    prev = jnp.concatenate([jnp.full((1,), -1, i32), chunk_nb[:-1]])
    chunk_first = (chunk_nb != prev).astype(i32)
    vcnt = jnp.clip(cnt[chunk_nb] - (cidx - chunk_start[chunk_nb]) * EB,
                    0, EB).astype(i32)

    slot = chunk_start[bucket] * EB + rank
    slotm = jnp.concatenate([slot.astype(i32),
                             jnp.full((EPAD - NEDGES,), S - 1, i32)])
    dstm = jnp.concatenate([dst, jnp.zeros((EPAD - NEDGES,), i32)])
    srcm = jnp.concatenate([src, jnp.zeros((EPAD - NEDGES,), i32)])
    kr = EPAD // SC_K
    slotm = slotm.reshape(kr, SC_K)
    dstm = dstm.reshape(kr, SC_K)
    srcm = srcm.reshape(kr, SC_K)

    w1t = emb2_w[:, :H].T
    w2t = emb2_w[:, H:].T
    d1t = dp1_w.T
    d2t = dp2_w.T
    d3t = dp3_w.T
    dbias = jnp.stack([dp1_b, dp2_b, dp3_b])
    eb2b = emb2_b.reshape(1, H)
    perm = (jnp.arange(3 * H) % H) * 3 + jnp.arange(3 * H) // H
    ls1t = ls1_w.T[:, perm]
    ls1b = ls1_b[perm].reshape(1, 3 * H)
    ls0t = ls0_w.T
    ls0b = ls0_b.reshape(1, 2 * H)
    lt0t = lt0_w.T
    lt1t = lt1_w.T
    lt2t = lt2_w.T

    dloc_e = (dst - bucket * NB).astype(f32)
    feat = jnp.concatenate([
        edge_attr.astype(f32),
        edge_vec_norm.astype(f32),
        edge_weight.astype(f32)[:, None],
        dloc_e[:, None],
        jnp.zeros((NEDGES, FEATW - NRBF - 5), f32),
    ], axis=1)
    feat = jnp.concatenate([feat, jnp.zeros((EPAD - NEDGES, FEATW), f32)])

    z_f = jnp.concatenate([z.astype(f32),
                           jnp.full((NPAD - NNODES,), -1.0, f32)])

    zw1, zw2 = _node_precompute(z_f, emb_w.astype(f32), w1t, w2t)

    feat_s, zij_s = _sc_scatter(feat, slotm, dstm, srcm, zw1, zw2)

    t_acc = _edge_accumulate(feat_s, zij_s, chunk_nb, chunk_first, vcnt,
                             d1t, d2t, d3t, dbias, eb2b)

    out9 = _final_stage(t_acc, ln_g.reshape(1, H), ln_b.reshape(1, H),
                        ls0t, ls0b, ls1t, ls1b, lt0t, lt1t, lt2t)

    out = out9[:, :NNODES, :].transpose(1, 2, 0).reshape(NNODES, H, 3, 3)
    return out
